# baseline (speedup 1.0000x reference)
import functools

import jax
import jax.numpy as jnp
from jax import lax
from jax.experimental import pallas as pl
from jax.experimental.pallas import tpu as pltpu
from jax.experimental.pallas import tpu_sc as plsc

_PSIZES = (41, 37, 31, 23)
_D = 16
_NT = 4
_B = 16384
_NC = 2
_NS = 16
_NW = _NC * _NS
_BPW = _B // _NW
_CHUNK = 128
_NCHUNK = _BPW // _CHUNK
_L = 16


def _body(uid_hbm, w0, w1, w2, w3, out_hbm, uid_v, idx_v, rows_v, sem):
    tables = (w0, w1, w2, w3)
    wid = lax.axis_index("s") * _NC + lax.axis_index("c")
    base = wid * _BPW

    with jax.named_scope("uid_stage"):
        pltpu.sync_copy(uid_hbm.at[pl.ds(base, _BPW)], uid_v)

    def mod_step(g, carry):
        u = uid_v[pl.ds(g * _L, _L)]
        uf = u.astype(jnp.float32)
        for t, p in enumerate(_PSIZES):
            q = (uf * (1.0 / p)).astype(jnp.int32).astype(jnp.float32)
            r = uf - q * float(p)
            r = jnp.where(r < 0.0, r + p, r)
            r = jnp.where(r >= p, r - p, r)
            idx_v[t, pl.ds(g * _L, _L)] = r.astype(jnp.int32)
        return carry

    with jax.named_scope("mod_compute"):
        lax.fori_loop(0, _BPW // _L, mod_step, 0)

    def fire(t):
        return [
            pltpu.async_copy(
                tables[t].at[idx_v.at[t, pl.ds(c * _CHUNK, _CHUNK)]],
                rows_v.at[t, pl.ds(c * _CHUNK, _CHUNK)],
                sem,
            )
            for c in range(_NCHUNK)
        ]

    with jax.named_scope("gather_and_write"):
        pending = fire(0)
        for t in range(_NT):
            nxt = fire(t + 1) if t + 1 < _NT else []
            with jax.named_scope(f"wait{t}"):
                for cp in pending:
                    cp.wait()
            with jax.named_scope(f"write{t}"):
                pltpu.sync_copy(rows_v.at[t], out_hbm.at[pl.ds(base, _BPW), t])
            pending = nxt


@functools.partial(
    pl.kernel,
    out_type=jax.ShapeDtypeStruct((_B, _NT, _D), jnp.float32),
    mesh=plsc.VectorSubcoreMesh(core_axis_name="c", subcore_axis_name="s"),
    scratch_types=[
        pltpu.VMEM((_BPW,), jnp.int32),
        pltpu.VMEM((_NT, _BPW), jnp.int32),
        pltpu.VMEM((_NT, _BPW, _D), jnp.float32),
        pltpu.SemaphoreType.DMA,
    ],
    compiler_params=pltpu.CompilerParams(use_tc_tiling_on_sc=False),
)
def _sc_lookup(uid_hbm, w0, w1, w2, w3, out_hbm, uid_v, idx_v, rows_v, sem):
    _body(uid_hbm, w0, w1, w2, w3, out_hbm, uid_v, idx_v, rows_v, sem)


def kernel(user_ids, W0, W1, W2, W3):
    out = _sc_lookup(user_ids.astype(jnp.int32), W0, W1, W2, W3)
    return out.reshape(_B, _NT * _D)


# speedup vs baseline: 1.8639x; 1.8639x over previous
import functools

import jax
import jax.numpy as jnp
from jax import lax
from jax.experimental import pallas as pl
from jax.experimental.pallas import tpu as pltpu
from jax.experimental.pallas import tpu_sc as plsc

_PSIZES = (41, 37, 31, 23)
_D = 16
_NT = 4
_B = 16384
_NC = 2
_NS = 16
_NW = _NC * _NS
_BPW = _B // _NW
_L = 16


def _body(uid_hbm, w0, w1, w2, w3, out_hbm, uid_v, w_v, out_v):
    tables = (w0, w1, w2, w3)
    wid = lax.axis_index("s") * _NC + lax.axis_index("c")
    base = wid * _BPW

    with jax.named_scope("stage"):
        pltpu.sync_copy(uid_hbm.at[pl.ds(base, _BPW)], uid_v)
        for t in range(_NT):
            pltpu.sync_copy(tables[t], w_v[t])

    lanes = lax.iota(jnp.int32, _L)

    def g_step(g, carry):
        u = uid_v[pl.ds(g * _L, _L)]
        uf = u.astype(jnp.float32)
        rows = g * _L + lanes
        for t, p in enumerate(_PSIZES):
            q = (uf * (1.0 / p)).astype(jnp.int32).astype(jnp.float32)
            r = uf - q * float(p)
            r = jnp.where(r < 0.0, r + p, r)
            r = jnp.where(r >= p, r - p, r)
            idx = r.astype(jnp.int32)
            for c in range(_D):
                cc = jnp.full((_L,), c, jnp.int32)
                kk = jnp.full((_L,), t * _D + c, jnp.int32)
                vals = plsc.load_gather(w_v[t], [idx, cc])
                plsc.store_scatter(out_v, [rows, kk], vals)
        return carry

    with jax.named_scope("lookup"):
        lax.fori_loop(0, _BPW // _L, g_step, 0)

    with jax.named_scope("writeback"):
        pltpu.sync_copy(out_v, out_hbm.at[pl.ds(base, _BPW)])


@functools.partial(
    pl.kernel,
    out_type=jax.ShapeDtypeStruct((_B, _NT * _D), jnp.float32),
    mesh=plsc.VectorSubcoreMesh(core_axis_name="c", subcore_axis_name="s"),
    scratch_types=[
        pltpu.VMEM((_BPW,), jnp.int32),
        tuple(pltpu.VMEM((p, _D), jnp.float32) for p in _PSIZES),
        pltpu.VMEM((_BPW, _NT * _D), jnp.float32),
    ],
    compiler_params=pltpu.CompilerParams(
        use_tc_tiling_on_sc=False, needs_layout_passes=False
    ),
)
def _sc_lookup(uid_hbm, w0, w1, w2, w3, out_hbm, uid_v, w_v, out_v):
    _body(uid_hbm, w0, w1, w2, w3, out_hbm, uid_v, w_v, out_v)


def kernel(user_ids, W0, W1, W2, W3):
    return _sc_lookup(user_ids.astype(jnp.int32), W0, W1, W2, W3)


# speedup vs baseline: 2.1177x; 1.1362x over previous
import functools

import jax
import jax.numpy as jnp
from jax import lax
from jax.experimental import pallas as pl
from jax.experimental.pallas import tpu as pltpu
from jax.experimental.pallas import tpu_sc as plsc

_PSIZES = (41, 37, 31, 23)
_D = 16
_NT = 4
_B = 16384
_NC = 2
_NS = 16
_NW = _NC * _NS
_BPW = _B // _NW
_L = 16


def _body(uid_hbm, w0, w1, w2, w3, out_hbm, uid_v, w_v, out_v):
    tables = (w0, w1, w2, w3)
    wid = lax.axis_index("s") * _NC + lax.axis_index("c")
    base = wid * _BPW

    with jax.named_scope("stage"):
        pltpu.sync_copy(uid_hbm.at[pl.ds(base, _BPW)], uid_v)
        for t in range(_NT):
            pltpu.sync_copy(tables[t], w_v[t])

    lanes = lax.iota(jnp.int32, _L)

    def g_step(i):
        g = i // _L
        u = uid_v[pl.ds(i, _L)]
        uf = u.astype(jnp.float32)
        rows = i + lanes
        for t, p in enumerate(_PSIZES):
            q = (uf * (1.0 / p)).astype(jnp.int32).astype(jnp.float32)
            r = uf - q * float(p)
            r = jnp.where(r < 0.0, r + p, r)
            r = jnp.where(r >= p, r - p, r)
            idx = r.astype(jnp.int32)
            for c in range(_D):
                cc = jnp.full((_L,), c, jnp.int32)
                kk = jnp.full((_L,), t * _D + c, jnp.int32)
                vals = plsc.load_gather(w_v[t], [idx, cc])
                plsc.store_scatter(out_v, [rows, kk], vals)
        del g

    with jax.named_scope("lookup"):
        plsc.parallel_loop(0, _BPW, step=_L, unroll=2)(g_step)

    with jax.named_scope("writeback"):
        pltpu.sync_copy(out_v, out_hbm.at[pl.ds(base, _BPW)])


@functools.partial(
    pl.kernel,
    out_type=jax.ShapeDtypeStruct((_B, _NT * _D), jnp.float32),
    mesh=plsc.VectorSubcoreMesh(core_axis_name="c", subcore_axis_name="s"),
    scratch_types=[
        pltpu.VMEM((_BPW,), jnp.int32),
        tuple(pltpu.VMEM((p, _D), jnp.float32) for p in _PSIZES),
        pltpu.VMEM((_BPW, _NT * _D), jnp.float32),
    ],
    compiler_params=pltpu.CompilerParams(
        use_tc_tiling_on_sc=False,
        needs_layout_passes=False,
        disable_bounds_checks=True,
    ),
)
def _sc_lookup(uid_hbm, w0, w1, w2, w3, out_hbm, uid_v, w_v, out_v):
    _body(uid_hbm, w0, w1, w2, w3, out_hbm, uid_v, w_v, out_v)


def kernel(user_ids, W0, W1, W2, W3):
    return _sc_lookup(user_ids.astype(jnp.int32), W0, W1, W2, W3)


# speedup vs baseline: 2.2137x; 1.0453x over previous
import functools

import jax
import jax.numpy as jnp
from jax import lax
from jax.experimental import pallas as pl
from jax.experimental.pallas import tpu as pltpu
from jax.experimental.pallas import tpu_sc as plsc

_PSIZES = (41, 37, 31, 23)
_D = 16
_NT = 4
_B = 16384
_NC = 2
_NS = 16
_NW = _NC * _NS
_BPW = _B // _NW
_L = 16
_PR = 48
_PCOL = 128


def _body(uid_hbm, w0, w1, w2, w3, out_hbm, uid_v, w_v, out_v):
    tables = (w0, w1, w2, w3)
    wid = lax.axis_index("s") * _NC + lax.axis_index("c")
    base = wid * _BPW

    with jax.named_scope("stage"):
        pltpu.sync_copy(uid_hbm.at[pl.ds(base, _BPW)], uid_v)
        for t in range(_NT):
            pltpu.sync_copy(tables[t], w_v[t])

    lanes = lax.iota(jnp.int32, _L)

    def g_step(i):
        u = uid_v[pl.ds(i, _L)]
        uf = u.astype(jnp.float32)
        rows = i + lanes
        for t, p in enumerate(_PSIZES):
            q = (uf * (1.0 / p)).astype(jnp.int32).astype(jnp.float32)
            r = uf - q * float(p)
            r = jnp.where(r < 0.0, r + p, r)
            r = jnp.where(r >= p, r - p, r)
            idx = r.astype(jnp.int32)
            for c in range(_D):
                cc = jnp.full((_L,), c, jnp.int32)
                kk = jnp.full((_L,), t * _D + c, jnp.int32)
                vals = plsc.load_gather(w_v[t], [idx, cc])
                plsc.store_scatter(out_v, [rows, kk], vals)

    with jax.named_scope("lookup"):
        plsc.parallel_loop(0, _BPW, step=_L, unroll=4)(g_step)

    with jax.named_scope("writeback"):
        pltpu.sync_copy(out_v, out_hbm.at[pl.ds(base, _BPW)])


@functools.partial(
    pl.kernel,
    out_type=jax.ShapeDtypeStruct((_B, _NT * _D), jnp.float32),
    mesh=plsc.VectorSubcoreMesh(core_axis_name="c", subcore_axis_name="s"),
    scratch_types=[
        pltpu.VMEM((_BPW,), jnp.int32),
        tuple(pltpu.VMEM((_PR, _PCOL), jnp.float32) for _ in _PSIZES),
        pltpu.VMEM((_BPW, _NT * _D), jnp.float32),
    ],
    compiler_params=pltpu.CompilerParams(
        use_tc_tiling_on_sc=True,
        needs_layout_passes=False,
        disable_bounds_checks=True,
    ),
)
def _sc_lookup(uid_hbm, w0, w1, w2, w3, out_hbm, uid_v, w_v, out_v):
    _body(uid_hbm, w0, w1, w2, w3, out_hbm, uid_v, w_v, out_v)


def kernel(user_ids, W0, W1, W2, W3):
    pads = [
        jnp.pad(w, ((0, _PR - p), (0, _PCOL - _D)))
        for w, p in zip((W0, W1, W2, W3), _PSIZES)
    ]
    return _sc_lookup(user_ids.astype(jnp.int32), *pads)


# speedup vs baseline: 3.3229x; 1.5011x over previous
import functools

import jax
import jax.numpy as jnp
from jax import lax
from jax.experimental import pallas as pl
from jax.experimental.pallas import tpu as pltpu
from jax.experimental.pallas import tpu_sc as plsc

_PSIZES = (41, 37, 31, 23)
_D = 16
_NT = 4
_B = 16384
_NC = 2
_NS = 16
_NW = _NC * _NS
_BPW = _B // _NW
_L = 16
_PR = 48
_PCOL = 128

_GATHER_DNUMS = lax.GatherDimensionNumbers(
    offset_dims=(), collapsed_slice_dims=(0,), start_index_map=(0,)
)


def _bcast_lane(vec, j):
    jj = jnp.full((_L, 1), j, jnp.int32)
    return lax.gather(
        vec, jj, _GATHER_DNUMS, (1,),
        mode=lax.GatherScatterMode.PROMISE_IN_BOUNDS,
    )


def _body(uid_hbm, w0, w1, w2, w3, out_hbm, uid_v, w_v, out_v, sem):
    tables = (w0, w1, w2, w3)
    wid = lax.axis_index("s") * _NC + lax.axis_index("c")
    base = wid * _BPW

    with jax.named_scope("stage"):
        copies = [pltpu.async_copy(uid_hbm.at[pl.ds(base, _BPW)], uid_v, sem)]
        for t in range(_NT):
            copies.append(pltpu.async_copy(tables[t], w_v[t], sem))
        for cp in copies:
            cp.wait()

    lanes = lax.iota(jnp.int32, _L)

    def g_step(i):
        u = uid_v[pl.ds(i, _L)]
        uf = u.astype(jnp.float32)
        idxs = []
        for t, p in enumerate(_PSIZES):
            q = (uf * (1.0 / p)).astype(jnp.int32).astype(jnp.float32)
            r = uf - q * float(p)
            r = jnp.where(r < 0.0, r + p, r)
            r = jnp.where(r >= p, r - p, r)
            idxs.append(r.astype(jnp.int32))
        for j in range(_L):
            for t in range(_NT):
                row = _bcast_lane(idxs[t], j)
                vals = plsc.load_gather(w_v[t], [row, lanes])
                out_v[i + j, pl.ds(t * _D, _D)] = vals

    with jax.named_scope("lookup"):
        plsc.parallel_loop(0, _BPW, step=_L, unroll=2)(g_step)

    with jax.named_scope("writeback"):
        pltpu.sync_copy(out_v, out_hbm.at[pl.ds(base, _BPW)])


@functools.partial(
    pl.kernel,
    out_type=jax.ShapeDtypeStruct((_B, _NT * _D), jnp.float32),
    mesh=plsc.VectorSubcoreMesh(core_axis_name="c", subcore_axis_name="s"),
    scratch_types=[
        pltpu.VMEM((_BPW,), jnp.int32),
        tuple(pltpu.VMEM((_PR, _PCOL), jnp.float32) for _ in _PSIZES),
        pltpu.VMEM((_BPW, _NT * _D), jnp.float32),
        pltpu.SemaphoreType.DMA,
    ],
    compiler_params=pltpu.CompilerParams(
        use_tc_tiling_on_sc=True,
        needs_layout_passes=False,
        disable_bounds_checks=True,
    ),
)
def _sc_lookup(uid_hbm, w0, w1, w2, w3, out_hbm, uid_v, w_v, out_v, sem):
    _body(uid_hbm, w0, w1, w2, w3, out_hbm, uid_v, w_v, out_v, sem)


def kernel(user_ids, W0, W1, W2, W3):
    pads = [
        jnp.pad(w, ((0, _PR - p), (0, _PCOL - _D)))
        for w, p in zip((W0, W1, W2, W3), _PSIZES)
    ]
    return _sc_lookup(user_ids.astype(jnp.int32), *pads)
